# Initial kernel scaffold; baseline (speedup 1.0000x reference)
import jax
import jax.numpy as jnp
from jax.experimental import pallas as pl


def kernel(x, index, weight, bias, decay_value, cache):
    raise NotImplementedError("write your pallas kernel here")


# speedup vs baseline: 1.0650x; 1.0650x over previous
import jax
import jax.numpy as jnp
from jax import lax
from jax.experimental import pallas as pl
from jax.experimental.pallas import tpu as pltpu

_DIM = 8192
_BM = 512


def _body(idx_ref, dv_ref, w_ref, b_ref, cache_ref, x_ref, o_ref):
    idx = idx_ref[0]
    col = lax.broadcasted_iota(jnp.int32, (1, _DIM), 1)
    w = jnp.sum(jnp.where(col == idx, w_ref[...], 0.0))
    b = jnp.sum(jnp.where(col == idx, b_ref[...], 0.0))
    dv = jnp.clip(dv_ref[0], 0.9, 1.0)
    c = dv * cache_ref[...] + b
    o_ref[...] = w * x_ref[...] + c


def kernel(x, index, weight, bias, decay_value, cache):
    n, d = x.shape
    idx = jnp.asarray(index, jnp.int32).reshape(1)
    grid = (n // _BM,)
    return pl.pallas_call(
        _body,
        grid=grid,
        in_specs=[
            pl.BlockSpec(memory_space=pltpu.SMEM),
            pl.BlockSpec(memory_space=pltpu.SMEM),
            pl.BlockSpec((1, _DIM), lambda i: (0, 0)),
            pl.BlockSpec((1, _DIM), lambda i: (0, 0)),
            pl.BlockSpec((1, d), lambda i: (0, 0)),
            pl.BlockSpec((_BM, d), lambda i: (i, 0)),
        ],
        out_specs=pl.BlockSpec((_BM, d), lambda i: (i, 0)),
        out_shape=jax.ShapeDtypeStruct((n, d), jnp.float32),
    )(idx, decay_value, weight, bias.reshape(1, _DIM), cache.reshape(1, d), x)
